# baseline (speedup 1.0000x reference)
import math

import jax
import jax.numpy as jnp
from jax import lax
from jax.experimental import pallas as pl
from jax.experimental.pallas import tpu as pltpu
from jax.experimental.pallas import tpu_sc as plsc

D_MODEL = 64
SCALE = math.sqrt(D_MODEL)
NC, NS = 2, 16
NW = NC * NS
LANES = 16
CHUNK = 100
NBUF = 4


def _embed_body(x_hbm, lut_hbm, out_hbm, idx_v, grows, wrows, gsem, wsem):
    wid = lax.axis_index("s") * NC + lax.axis_index("c")
    n_chunks = x_hbm.shape[1]
    rows_per_w = n_chunks // 2
    row_base = wid * rows_per_w

    pltpu.sync_copy(x_hbm.at[wid], idx_v)

    for b in range(NBUF):
        pltpu.async_copy(lut_hbm.at[idx_v.at[b]], grows.at[b], gsem.at[b])

    def outer(r2, carry):
        for rr in range(2):
            for h in range(2):
                b = 2 * rr + h
                c = 4 * r2 + b
                row = row_base + 2 * r2 + rr

                pltpu.make_async_copy(
                    lut_hbm.at[idx_v.at[c]], grows.at[b], gsem.at[b]).wait()

                @pl.when(c >= NBUF)
                def _():
                    pltpu.make_async_copy(
                        wrows.at[b], out_hbm.at[0, pl.ds(0, CHUNK), :],
                        wsem.at[b]).wait()

                @plsc.parallel_loop(0, CHUNK, step=2, unroll=2)
                def _(i):
                    for r in range(2):
                        for j in range(D_MODEL // LANES):
                            sl = pl.ds(j * LANES, LANES)
                            wrows[b, i + r, sl] = grows[b, i + r, sl] * SCALE

                pltpu.async_copy(
                    wrows.at[b],
                    out_hbm.at[row, pl.ds(h * CHUNK, CHUNK), :],
                    wsem.at[b])

                @pl.when(c + NBUF < n_chunks)
                def _():
                    pltpu.async_copy(
                        lut_hbm.at[idx_v.at[c + NBUF]], grows.at[b],
                        gsem.at[b])
        return carry

    lax.fori_loop(0, n_chunks // NBUF, outer, 0)

    for b in range(NBUF):
        pltpu.make_async_copy(
            wrows.at[b], out_hbm.at[0, pl.ds(0, CHUNK), :], wsem.at[b]).wait()


def kernel(x, lut):
    S, T = x.shape
    n_chunks = S * T // (NW * CHUNK)
    xf = x.reshape(NW, n_chunks, CHUNK)
    k = pl.kernel(
        _embed_body,
        out_type=jax.ShapeDtypeStruct((S, T, D_MODEL), jnp.float32),
        mesh=plsc.VectorSubcoreMesh(core_axis_name="c", subcore_axis_name="s"),
        scratch_types=[
            pltpu.VMEM((n_chunks, CHUNK), jnp.int32),
            pltpu.VMEM((NBUF, CHUNK, D_MODEL), jnp.float32),
            pltpu.VMEM((NBUF, CHUNK, D_MODEL), jnp.float32),
            pltpu.SemaphoreType.DMA((NBUF,)),
            pltpu.SemaphoreType.DMA((NBUF,)),
        ],
        compiler_params=pltpu.CompilerParams(use_tc_tiling_on_sc=False),
    )
    return k(xf, lut)
